# baseline (device time: 75831 ns/iter reference)
import jax
import jax.numpy as jnp
from jax import lax
from jax.experimental import pallas as pl
from jax.experimental.pallas import tpu as pltpu

N_DEV = 4
SQ, D = 512, 1024
HL, DH, SKV = 8, 128, 2048
SCALE = 0.08838834764831843


def kernel(x, Wq, Wo, K_ext, V_ext):
    xb = x.reshape(SQ, D)
    kb = K_ext.reshape(SKV, HL * DH)
    vb = V_ext.reshape(SKV, HL * DH)

    def body(x_ref, wq_ref, wo_ref, k_ref, v_ref, out_ref,
             comm_ref, send_sems, recv_sems):
        my = lax.axis_index("i")
        left = lax.rem(my + N_DEV - 1, N_DEV)
        right = lax.rem(my + 1, N_DEV)

        xv = x_ref[:, :].astype(jnp.bfloat16)
        wq = wq_ref[:, :].astype(jnp.bfloat16)
        q = lax.dot_general(
            xv, wq, (((1,), (0,)), ((), ())),
            preferred_element_type=jnp.float32,
        ).astype(jnp.bfloat16)
        kv = k_ref[:, :].astype(jnp.bfloat16)
        vv = v_ref[:, :].astype(jnp.bfloat16)

        outs = []
        for h in range(HL):
            sl = slice(h * DH, (h + 1) * DH)
            s = lax.dot_general(
                q[:, sl], kv[:, sl], (((1,), (1,)), ((), ())),
                preferred_element_type=jnp.float32,
            ) * SCALE
            m = jnp.max(s, axis=1, keepdims=True)
            p = jnp.exp(s - m)
            l = jnp.sum(p, axis=1, keepdims=True)
            o = lax.dot_general(
                p.astype(jnp.bfloat16), vv[:, sl], (((1,), (0,)), ((), ())),
                preferred_element_type=jnp.float32,
            )
            outs.append((o / l).astype(jnp.bfloat16))
        ao = jnp.concatenate(outs, axis=1)
        wo = wo_ref[:, :].astype(jnp.bfloat16)
        partial = lax.dot_general(
            ao, wo, (((1,), (0,)), ((), ())),
            preferred_element_type=jnp.float32,
        )

        comm_ref[0, :, :] = partial.astype(jnp.bfloat16)

        barrier_sem = pltpu.get_barrier_semaphore()
        for nbr in (left, right):
            pl.semaphore_signal(
                barrier_sem, inc=1,
                device_id=(nbr,), device_id_type=pl.DeviceIdType.MESH,
            )
        pl.semaphore_wait(barrier_sem, 2)

        acc = partial
        for h in range(N_DEV - 1):
            rdma = pltpu.make_async_remote_copy(
                src_ref=comm_ref.at[h],
                dst_ref=comm_ref.at[h + 1],
                send_sem=send_sems.at[h],
                recv_sem=recv_sems.at[h],
                device_id=(right,),
                device_id_type=pl.DeviceIdType.MESH,
            )
            rdma.start()
            rdma.wait()
            acc = acc + comm_ref[h + 1, :, :].astype(jnp.float32)
        out_ref[:, :] = acc

    out = pl.pallas_call(
        body,
        out_shape=jax.ShapeDtypeStruct((SQ, D), jnp.float32),
        in_specs=[pl.BlockSpec(memory_space=pltpu.VMEM)] * 5,
        out_specs=pl.BlockSpec(memory_space=pltpu.VMEM),
        scratch_shapes=[
            pltpu.VMEM((N_DEV, SQ, D), jnp.bfloat16),
            pltpu.SemaphoreType.DMA((N_DEV - 1,)),
            pltpu.SemaphoreType.DMA((N_DEV - 1,)),
        ],
        compiler_params=pltpu.CompilerParams(collective_id=0),
    )(xb, Wq, Wo, kb, vb)
    return out.reshape(1, SQ, D)
